# Initial kernel scaffold; baseline (speedup 1.0000x reference)
import functools

import jax
import jax.numpy as jnp
from jax import lax
from jax.experimental import pallas as pl
from jax.experimental.pallas import tpu as pltpu
from jax.experimental.pallas import tpu_sc as plsc

N = 10000
E = 320000
D = 128

NW = 32
CK = 128
CH = 80
PE = NW * CH * CK
NP = 10240
RPT = NP // 16


_sc_mesh = plsc.VectorSubcoreMesh(core_axis_name="c", subcore_axis_name="s")


@functools.partial(
    pl.kernel,
    out_type=jax.ShapeDtypeStruct((2, NP, D), jnp.float32),
    mesh=_sc_mesh,
    scratch_types=[
        pltpu.VMEM((CH, CK), jnp.int32),
        pltpu.VMEM((CH, CK), jnp.int32),
        pltpu.VMEM((CK, D), jnp.float32),
        pltpu.VMEM_SHARED((NP, D), jnp.float32),
        pltpu.SemaphoreType.DMA,
    ],
)
def _sc_segment(h_hbm, src_hbm, dst_hbm, out_hbm, sidx, didx, rows, acc, sem):
    c = lax.axis_index("c")
    s = lax.axis_index("s")
    wid = c * 16 + s
    pltpu.sync_copy(h_hbm.at[pl.ds(s * RPT, RPT)], acc.at[pl.ds(s * RPT, RPT)])
    pltpu.sync_copy(src_hbm.at[wid], sidx)
    pltpu.sync_copy(dst_hbm.at[wid], didx)
    plsc.subcore_barrier()

    def body(j, carry):
        pltpu.async_copy(h_hbm.at[sidx.at[j]], rows, sem).wait()
        pltpu.sync_copy(rows, acc.at[didx.at[j]], add=True)
        return carry

    lax.fori_loop(0, CH, body, 0)
    plsc.subcore_barrier()
    pltpu.sync_copy(acc.at[pl.ds(s * RPT, RPT)], out_hbm.at[c, pl.ds(s * RPT, RPT)])


def _dense_body(p0, p1, h, w1, g1, b1, w2, g2, b2, xo, pool):
    mask = lax.broadcasted_iota(jnp.int32, (NP, 1), 0) < N
    x = jnp.where(mask, p0[...] + p1[...] - h[...], 0.0)
    t = jnp.dot(x, w1[...], preferred_element_type=jnp.float32)
    mu = jnp.sum(t, axis=0, keepdims=True) * (1.0 / N)
    d = jnp.where(mask, t - mu, 0.0)
    var = jnp.sum(d * d, axis=0, keepdims=True) * (1.0 / N)
    y = g1[...] * d * jax.lax.rsqrt(var + 1e-5) + b1[...]
    y = jnp.where(mask, jnp.maximum(y, 0.0), 0.0)
    u = jnp.dot(y, w2[...], preferred_element_type=jnp.float32)
    mu2 = jnp.sum(u, axis=0, keepdims=True) * (1.0 / N)
    d2 = jnp.where(mask, u - mu2, 0.0)
    var2 = jnp.sum(d2 * d2, axis=0, keepdims=True) * (1.0 / N)
    z = g2[...] * d2 * jax.lax.rsqrt(var2 + 1e-5) + b2[...]
    z = jnp.where(mask, jnp.maximum(z, 0.0), 0.0)
    xo[...] = z
    pool[...] = jnp.sum(z, axis=0, keepdims=True)


_dense = pl.pallas_call(
    _dense_body,
    out_shape=(
        jax.ShapeDtypeStruct((NP, D), jnp.float32),
        jax.ShapeDtypeStruct((1, D), jnp.float32),
    ),
)


def _layer(h_pad, src3, dst3, W1, g1, b1, W2, bng, bnb):
    p = _sc_segment(h_pad, src3, dst3)
    return _dense(p[0], p[1], h_pad,
                  W1, g1.reshape(1, D), b1.reshape(1, D),
                  W2, bng.reshape(1, D), bnb.reshape(1, D))


def kernel(h, edge_index, W1_0, g1_0, b1_0, W2_0, bng_0, bnb_0,
           W1_1, g1_1, b1_1, W2_1, bng_1, bnb_1):
    pad = PE - E
    src3 = jnp.concatenate(
        [edge_index[0], jnp.zeros((pad,), jnp.int32)]).reshape(NW, CH, CK)
    dst3 = jnp.concatenate(
        [edge_index[1], jnp.full((pad,), N, jnp.int32)]).reshape(NW, CH, CK)
    h_pad = jnp.pad(h, ((0, NP - N), (0, 0)))
    h1, p0 = _layer(h_pad, src3, dst3, W1_0, g1_0, b1_0, W2_0, bng_0, bnb_0)
    h2, p1 = _layer(h1, src3, dst3, W1_1, g1_1, b1_1, W2_1, bng_1, bnb_1)
    return h2[:N], jnp.concatenate([p0, p1], axis=1)


# speedup vs baseline: 3.0627x; 3.0627x over previous
import functools

import jax
import jax.numpy as jnp
from jax import lax
from jax.experimental import pallas as pl
from jax.experimental.pallas import tpu as pltpu
from jax.experimental.pallas import tpu_sc as plsc

N = 10000
E = 320000
D = 128

NW = 32
CK = 128
CH = 80
PE = NW * CH * CK
NP = 10240
RPT = NP // 16


@functools.cache
def _make_sc_segment():
    mesh = plsc.VectorSubcoreMesh(core_axis_name="c", subcore_axis_name="s")
    return pl.kernel(
        _sc_segment_body,
        out_type=jax.ShapeDtypeStruct((2, NP, D), jnp.float32),
        mesh=mesh,
        scratch_types=[
            pltpu.VMEM((CH, CK), jnp.int32),
            pltpu.VMEM((CH, CK), jnp.int32),
            pltpu.VMEM((CK, D), jnp.float32),
            pltpu.VMEM_SHARED((NP, D), jnp.float32),
            pltpu.SemaphoreType.DMA,
        ],
    )


def _sc_segment_body(h_hbm, src_hbm, dst_hbm, out_hbm, sidx, didx, rows, acc, sem):
    c = lax.axis_index("c")
    s = lax.axis_index("s")
    wid = c * 16 + s
    pltpu.sync_copy(h_hbm.at[pl.ds(s * RPT, RPT)], acc.at[pl.ds(s * RPT, RPT)])
    pltpu.sync_copy(src_hbm.at[wid], sidx)
    pltpu.sync_copy(dst_hbm.at[wid], didx)
    plsc.subcore_barrier()

    def body(j, carry):
        pltpu.async_copy(h_hbm.at[sidx.at[j]], rows, sem).wait()
        pltpu.sync_copy(rows, acc.at[didx.at[j]], add=True)
        return carry

    lax.fori_loop(0, CH, body, 0)
    plsc.subcore_barrier()
    pltpu.sync_copy(acc.at[pl.ds(s * RPT, RPT)], out_hbm.at[c, pl.ds(s * RPT, RPT)])


def _dense_body(p0, p1, h, w1, g1, b1, w2, g2, b2, xo, pool):
    mask = lax.broadcasted_iota(jnp.int32, (NP, 1), 0) < N
    x = jnp.where(mask, p0[...] + p1[...] - h[...], 0.0)
    t = jnp.dot(x, w1[...], preferred_element_type=jnp.float32)
    mu = jnp.sum(t, axis=0, keepdims=True) * (1.0 / N)
    d = jnp.where(mask, t - mu, 0.0)
    var = jnp.sum(d * d, axis=0, keepdims=True) * (1.0 / N)
    y = g1[...] * d * jax.lax.rsqrt(var + 1e-5) + b1[...]
    y = jnp.where(mask, jnp.maximum(y, 0.0), 0.0)
    u = jnp.dot(y, w2[...], preferred_element_type=jnp.float32)
    mu2 = jnp.sum(u, axis=0, keepdims=True) * (1.0 / N)
    d2 = jnp.where(mask, u - mu2, 0.0)
    var2 = jnp.sum(d2 * d2, axis=0, keepdims=True) * (1.0 / N)
    z = g2[...] * d2 * jax.lax.rsqrt(var2 + 1e-5) + b2[...]
    z = jnp.where(mask, jnp.maximum(z, 0.0), 0.0)
    xo[...] = z
    pool[...] = jnp.sum(z, axis=0, keepdims=True)


_dense = pl.pallas_call(
    _dense_body,
    out_shape=(
        jax.ShapeDtypeStruct((NP, D), jnp.float32),
        jax.ShapeDtypeStruct((1, D), jnp.float32),
    ),
)


def _layer(h_pad, src3, dst3, W1, g1, b1, W2, bng, bnb):
    p = _make_sc_segment()(h_pad, src3, dst3)
    return _dense(p[0], p[1], h_pad,
                  W1, g1.reshape(1, D), b1.reshape(1, D),
                  W2, bng.reshape(1, D), bnb.reshape(1, D))


def kernel(h, edge_index, W1_0, g1_0, b1_0, W2_0, bng_0, bnb_0,
           W1_1, g1_1, b1_1, W2_1, bng_1, bnb_1):
    pad = PE - E
    src3 = jnp.concatenate(
        [edge_index[0], jnp.zeros((pad,), jnp.int32)]).reshape(NW, CH, CK)
    dst3 = jnp.concatenate(
        [edge_index[1], jnp.full((pad,), N, jnp.int32)]).reshape(NW, CH, CK)
    h_pad = jnp.pad(h, ((0, NP - N), (0, 0)))
    h1, p0 = _layer(h_pad, src3, dst3, W1_0, g1_0, b1_0, W2_0, bng_0, bnb_0)
    h2, p1 = _layer(h1, src3, dst3, W1_1, g1_1, b1_1, W2_1, bng_1, bnb_1)
    return h2[:N], jnp.concatenate([p0, p1], axis=1)


# speedup vs baseline: 3.8680x; 1.2629x over previous
import functools

import jax
import jax.numpy as jnp
from jax import lax
from jax.experimental import pallas as pl
from jax.experimental.pallas import tpu as pltpu
from jax.experimental.pallas import tpu_sc as plsc

N = 10000
E = 320000
D = 128

NW = 32
CK = 128
CH = 80
PE = NW * CH * CK
NP = 10240
RPT = NP // 16


NB = 2
NI = 4


@functools.cache
def _make_sc_segment():
    mesh = plsc.VectorSubcoreMesh(core_axis_name="c", subcore_axis_name="s")
    return pl.kernel(
        _sc_segment_body,
        out_type=jax.ShapeDtypeStruct((2, NP, D), jnp.float32),
        mesh=mesh,
        scratch_types=[
            [pltpu.VMEM((2, CK), jnp.int32) for _ in range(NI)],
            [pltpu.VMEM((CK, D), jnp.float32) for _ in range(NB)],
            pltpu.VMEM_SHARED((NP, D), jnp.float32),
            [pltpu.SemaphoreType.DMA for _ in range(NI)],
            [pltpu.SemaphoreType.DMA for _ in range(NB)],
            [pltpu.SemaphoreType.DMA for _ in range(NB)],
        ],
    )


def _sc_segment_body(h_hbm, eidx_hbm, out_hbm, idxs, rows, acc,
                     isems, gsems, ssems):
    c = lax.axis_index("c")
    s = lax.axis_index("s")
    wid = c * 16 + s
    pltpu.sync_copy(h_hbm.at[pl.ds(s * RPT, RPT)], acc.at[pl.ds(s * RPT, RPT)])
    plsc.subcore_barrier()

    def idx_issue(j, slot):
        pltpu.async_copy(eidx_hbm.at[wid, j], idxs[slot], isems[slot])

    def gather(j, islot, b):
        return pltpu.make_async_copy(
            h_hbm.at[idxs[islot].at[0]], rows[b], gsems[b])

    def scatter(islot, b):
        return pltpu.make_async_copy(
            rows[b], acc.at[idxs[islot].at[1]], ssems[b])

    idx_issue(0, 0)
    idx_issue(1, 1)
    pltpu.make_async_copy(eidx_hbm.at[wid, 0], idxs[0], isems[0]).wait()
    gather(0, 0, 0).start()

    def body(g, carry):
        for k in range(4):
            j = g * 4 + k
            b = k % 2
            o = 1 - b
            kn1 = (k + 1) % 4
            kn2 = (k + 2) % 4

            @pl.when(j >= 1)
            def _():
                scatter(kn1, o).wait()

            @pl.when(j + 1 < CH)
            def _():
                pltpu.make_async_copy(
                    eidx_hbm.at[wid, j + 1], idxs[kn1], isems[kn1]).wait()
                gather(j + 1, kn1, o).start()

            @pl.when(j + 2 < CH)
            def _():
                idx_issue(j + 2, kn2)

            gather(j, k, b).wait()
            scatter(k, b).start(add=True)
        return carry

    lax.fori_loop(0, CH // 4, body, 0)
    pltpu.make_async_copy(
        rows[(CH - 1) % 2], acc.at[idxs[(CH - 1) % 4].at[1]],
        ssems[(CH - 1) % 2]).wait()
    plsc.subcore_barrier()
    pltpu.sync_copy(acc.at[pl.ds(s * RPT, RPT)], out_hbm.at[c, pl.ds(s * RPT, RPT)])


def _dense_body(p0, p1, h, w1, g1, b1, w2, g2, b2, xo, pool):
    mask = lax.broadcasted_iota(jnp.int32, (NP, 1), 0) < N
    x = jnp.where(mask, p0[...] + p1[...] - h[...], 0.0)
    t = jnp.dot(x, w1[...], preferred_element_type=jnp.float32)
    mu = jnp.sum(t, axis=0, keepdims=True) * (1.0 / N)
    d = jnp.where(mask, t - mu, 0.0)
    var = jnp.sum(d * d, axis=0, keepdims=True) * (1.0 / N)
    y = g1[...] * d * jax.lax.rsqrt(var + 1e-5) + b1[...]
    y = jnp.where(mask, jnp.maximum(y, 0.0), 0.0)
    u = jnp.dot(y, w2[...], preferred_element_type=jnp.float32)
    mu2 = jnp.sum(u, axis=0, keepdims=True) * (1.0 / N)
    d2 = jnp.where(mask, u - mu2, 0.0)
    var2 = jnp.sum(d2 * d2, axis=0, keepdims=True) * (1.0 / N)
    z = g2[...] * d2 * jax.lax.rsqrt(var2 + 1e-5) + b2[...]
    z = jnp.where(mask, jnp.maximum(z, 0.0), 0.0)
    xo[...] = z
    pool[...] = jnp.sum(z, axis=0, keepdims=True)


_dense = pl.pallas_call(
    _dense_body,
    out_shape=(
        jax.ShapeDtypeStruct((NP, D), jnp.float32),
        jax.ShapeDtypeStruct((1, D), jnp.float32),
    ),
)


def _layer(h_pad, eidx, W1, g1, b1, W2, bng, bnb):
    p = _make_sc_segment()(h_pad, eidx)
    return _dense(p[0], p[1], h_pad,
                  W1, g1.reshape(1, D), b1.reshape(1, D),
                  W2, bng.reshape(1, D), bnb.reshape(1, D))


def kernel(h, edge_index, W1_0, g1_0, b1_0, W2_0, bng_0, bnb_0,
           W1_1, g1_1, b1_1, W2_1, bng_1, bnb_1):
    pad = PE - E
    src3 = jnp.concatenate(
        [edge_index[0], jnp.zeros((pad,), jnp.int32)]).reshape(NW, CH, CK)
    trash = N + (jnp.arange(pad, dtype=jnp.int32) % (NP - N))
    dst3 = jnp.concatenate([edge_index[1], trash]).reshape(NW, CH, CK)
    eidx = jnp.stack([src3, dst3], axis=2)
    h_pad = jnp.pad(h, ((0, NP - N), (0, 0)))
    h1, p0 = _layer(h_pad, eidx, W1_0, g1_0, b1_0, W2_0, bng_0, bnb_0)
    h2, p1 = _layer(h1, eidx, W1_1, g1_1, b1_1, W2_1, bng_1, bnb_1)
    return h2[:N], jnp.concatenate([p0, p1], axis=1)
